# Initial kernel scaffold; baseline (speedup 1.0000x reference)
import jax
import jax.numpy as jnp
from jax.experimental import pallas as pl


def kernel(logits, item_seq, target):
    raise NotImplementedError("write your pallas kernel here")


# speedup vs baseline: 2.5997x; 2.5997x over previous
import functools

import jax
import jax.numpy as jnp
from jax import lax
from jax.experimental import pallas as pl
from jax.experimental.pallas import tpu as pltpu
from jax.experimental.pallas import tpu_sc as plsc

_B = 1024
_N = 100000
_L = 50
_LANES = 128
_SUB = 64
_ROWS = 832
_BIG = 0x7FFFFFFF
_KS1 = 42
_KS2 = 42 ^ 0x1BD11BDA


def _threefry_pair(x0, x1):
    ks = (jnp.uint32(0), jnp.uint32(_KS1), jnp.uint32(_KS2))
    rots = ((13, 15, 26, 6), (17, 29, 16, 24))
    x0 = x0 + ks[0]
    x1 = x1 + ks[1]
    for i in range(5):
        for r in rots[i % 2]:
            x0 = x0 + x1
            x1 = (x1 << r) | (x1 >> (32 - r))
            x1 = x1 ^ x0
        x0 = x0 + ks[(i + 1) % 3]
        x1 = x1 + ks[(i + 2) % 3] + jnp.uint32(i + 1)
    return x0, x1


def _sample_body(iseq_ref, tgt_ref, neg_ref, mant_ref, j_ref):
    b = pl.program_id(0)

    @pl.when(b == 0)
    def _():
        sub = lax.broadcasted_iota(jnp.int32, (_ROWS, _LANES), 0)
        lane = lax.broadcasted_iota(jnp.int32, (_ROWS, _LANES), 1)
        j_ref[:, :] = sub * _LANES + lane

    base = (b * _N).astype(jnp.uint32)

    def chunk(i, carry):
        r0 = i * _SUB
        j = j_ref[pl.ds(r0, _SUB), :]
        x1 = j.astype(jnp.uint32) + base
        x0 = jnp.zeros((_SUB, _LANES), jnp.uint32)
        x0, x1 = _threefry_pair(x0, x1)
        mant = ((x0 ^ x1) >> 9).astype(jnp.int32)
        mant_ref[pl.ds(r0, _SUB), :] = jnp.where(j < _N, mant, jnp.int32(-1))
        return carry

    lax.fori_loop(0, _ROWS // _SUB, chunk, 0)

    tgt = tgt_ref[0, 0, 0]
    iseq = iseq_ref[0]

    def reduce_argmax():
        mv = mant_ref[:, :]
        m = jnp.max(mv)
        return jnp.min(jnp.where(mv == m, j_ref[:, :], jnp.int32(_BIG)))

    def is_banned(idx):
        return jnp.any(iseq == idx) | (idx == tgt)

    idx0 = reduce_argmax()

    def cond(c):
        return c[1]

    def body(c):
        idx_p, _ = c
        r = idx_p >> 7
        cc = idx_p & 127
        lane1 = lax.broadcasted_iota(jnp.int32, (1, _LANES), 1)
        row = mant_ref[pl.ds(r, 1), :]
        mant_ref[pl.ds(r, 1), :] = jnp.where(lane1 == cc, jnp.int32(-1), row)
        idx = reduce_argmax()
        return (idx, is_banned(idx))

    idx_f, _ = lax.while_loop(cond, body, (idx0, is_banned(idx0)))
    neg_ref[0, 0, 0] = idx_f


def _sample_neg(item_seq, target):
    iseq3 = item_seq.astype(jnp.int32).reshape(_B, 1, _L)
    tgt2 = target.astype(jnp.int32).reshape(_B, 1, 1)
    neg = pl.pallas_call(
        _sample_body,
        grid=(_B,),
        in_specs=[
            pl.BlockSpec((1, 1, _L), lambda b: (b, 0, 0)),
            pl.BlockSpec((1, 1, 1), lambda b: (b, 0, 0),
                         memory_space=pltpu.SMEM),
        ],
        out_specs=pl.BlockSpec((1, 1, 1), lambda b: (b, 0, 0),
                               memory_space=pltpu.SMEM),
        out_shape=jax.ShapeDtypeStruct((_B, 1, 1), jnp.int32),
        scratch_shapes=[
            pltpu.VMEM((_ROWS, _LANES), jnp.int32),
            pltpu.VMEM((_ROWS, _LANES), jnp.int32),
        ],
    )(iseq3, tgt2)
    return neg.reshape(_B)


_NW = 32
_BPW = _B // _NW


def _sc_gather(logits_flat, target, neg):
    mesh = plsc.VectorSubcoreMesh(core_axis_name="c", subcore_axis_name="s")

    @functools.partial(
        pl.kernel,
        mesh=mesh,
        out_type=[jax.ShapeDtypeStruct((_B,), jnp.float32),
                  jax.ShapeDtypeStruct((_B,), jnp.float32)],
        scratch_types=[
            pltpu.VMEM((_BPW,), jnp.int32),
            pltpu.VMEM((_BPW,), jnp.int32),
            pltpu.VMEM((_BPW,), jnp.float32),
            pltpu.VMEM((_BPW,), jnp.float32),
            pltpu.SemaphoreType.DMA,
            pltpu.SemaphoreType.DMA,
        ],
    )
    def gather_k(logits_hbm, tgt_hbm, neg_hbm, pos_out, negs_out,
                 ti_v, ni_v, pv, nv, s1, s2):
        wid = lax.axis_index("s") * 2 + lax.axis_index("c")
        base = wid * _BPW
        pltpu.sync_copy(tgt_hbm.at[pl.ds(base, _BPW)], ti_v)
        pltpu.sync_copy(neg_hbm.at[pl.ds(base, _BPW)], ni_v)
        for kk in range(_BPW // 16):
            sl = pl.ds(kk * 16, 16)
            rows = lax.iota(jnp.int32, 16) + (base + kk * 16)
            ti_v[sl] = rows * _N + ti_v[sl]
            ni_v[sl] = rows * _N + ni_v[sl]
        c1 = pltpu.async_copy(logits_hbm.at[ti_v], pv, s1)
        c2 = pltpu.async_copy(logits_hbm.at[ni_v], nv, s2)
        c1.wait()
        c2.wait()
        pltpu.sync_copy(pv, pos_out.at[pl.ds(base, _BPW)])
        pltpu.sync_copy(nv, negs_out.at[pl.ds(base, _BPW)])

    return gather_k(logits_flat, target.astype(jnp.int32), neg)


def _loss_body(pos_ref, neg_ref, out_ref):
    x = pos_ref[:, :] - neg_ref[:, :]
    ls = jnp.minimum(x, jnp.float32(0.0)) - jnp.log(1.0 + jnp.exp(-jnp.abs(x)))
    out_ref[0, 0] = -jnp.sum(ls) * jnp.float32(1.0 / _B)


def _loss(pos, negs):
    out = pl.pallas_call(
        _loss_body,
        in_specs=[pl.BlockSpec((8, 128), lambda: (0, 0)),
                  pl.BlockSpec((8, 128), lambda: (0, 0))],
        out_specs=pl.BlockSpec((1, 1), lambda: (0, 0),
                               memory_space=pltpu.SMEM),
        out_shape=jax.ShapeDtypeStruct((1, 1), jnp.float32),
    )(pos.reshape(8, 128), negs.reshape(8, 128))
    return out.reshape(())


def kernel(logits, item_seq, target):
    neg = _sample_neg(item_seq, target)
    pos_s, neg_s = _sc_gather(logits.reshape(-1), target, neg)
    return _loss(pos_s, neg_s)
